# baseline (speedup 1.0000x reference)
import functools

import jax
import jax.numpy as jnp
from jax import lax
from jax.experimental import pallas as pl
from jax.experimental.pallas import tpu as pltpu
from jax.experimental.pallas import tpu_sc as plsc

_B, _K = 16384, 64
_NC, _NS, _L = 2, 16, 16
_NW = _NC * _NS
_CW = _B // _NW
_NSUB = 4
_SUBW = _CW // _NSUB


def _sc_body(u_hbm, v_hbm, out_hbm, u_v, v_v, o_v, sems):
    wid = lax.axis_index("s") * _NC + lax.axis_index("c")
    base = wid * _CW
    copies = []
    for t in range(_NSUB):
        cu = pltpu.make_async_copy(
            u_hbm.at[:, pl.ds(base + t * _SUBW, _SUBW)],
            u_v.at[:, pl.ds(t * _SUBW, _SUBW)],
            sems.at[0, t],
        )
        cv = pltpu.make_async_copy(
            v_hbm.at[:, pl.ds(base + t * _SUBW, _SUBW)],
            v_v.at[:, pl.ds(t * _SUBW, _SUBW)],
            sems.at[1, t],
        )
        cu.start()
        cv.start()
        copies.append((cu, cv))

    zero = jnp.zeros((_L,), jnp.float32)
    for t in range(_NSUB):
        cu, cv = copies[t]
        cu.wait()
        cv.wait()

        def group_body(g, carry, t=t):
            col = pl.multiple_of(t * _SUBW + g * _L, _L)
            accs = [zero, zero, zero, zero]
            for r in range(_K):
                accs[r % 4] = (
                    accs[r % 4] + u_v[r, pl.ds(col, _L)] * v_v[r, pl.ds(col, _L)]
                )
            o_v[pl.ds(col, _L)] = (accs[0] + accs[1]) + (accs[2] + accs[3])
            return carry

        lax.fori_loop(0, _SUBW // _L, group_body, 0)

    pltpu.sync_copy(o_v, out_hbm.at[pl.ds(base, _CW)])


_sc_dot = functools.partial(
    pl.kernel,
    out_type=jax.ShapeDtypeStruct((_B,), jnp.float32),
    mesh=plsc.VectorSubcoreMesh(
        core_axis_name="c", subcore_axis_name="s", num_cores=_NC
    ),
    scratch_types=[
        pltpu.VMEM((_K, _CW), jnp.float32),
        pltpu.VMEM((_K, _CW), jnp.float32),
        pltpu.VMEM((_CW,), jnp.float32),
        pltpu.SemaphoreType.DMA((2, _NSUB)),
    ],
)(_sc_body)


def kernel(gu, gi):
    return _sc_dot(gu.T, gi.T)


# speedup vs baseline: 6.2959x; 6.2959x over previous
import jax
import jax.numpy as jnp
from jax.experimental import pallas as pl
from jax.experimental.pallas import tpu as pltpu

_B, _K = 16384, 64
_NCH = 8
_RS = _K // _NCH


def _body(u_hbm, v_hbm, out_ref, u_v, v_v, sems):
    copies = []
    for c in range(_NCH):
        cu = pltpu.make_async_copy(
            u_hbm.at[pl.ds(c * _RS, _RS), :],
            u_v.at[pl.ds(c * _RS, _RS), :],
            sems.at[0, c],
        )
        cv = pltpu.make_async_copy(
            v_hbm.at[pl.ds(c * _RS, _RS), :],
            v_v.at[pl.ds(c * _RS, _RS), :],
            sems.at[1, c],
        )
        cu.start()
        cv.start()
        copies.append((cu, cv))
    acc = None
    for c in range(_NCH):
        cu, cv = copies[c]
        cu.wait()
        cv.wait()
        sl = pl.ds(c * _RS, _RS)
        s = jnp.sum(u_v[sl, :] * v_v[sl, :], axis=0)
        acc = s if acc is None else acc + s
    out_ref[...] = acc.reshape(_B // 128, 128)


def kernel(gu, gi):
    out = pl.pallas_call(
        _body,
        in_specs=[
            pl.BlockSpec(memory_space=pltpu.HBM),
            pl.BlockSpec(memory_space=pltpu.HBM),
        ],
        out_specs=pl.BlockSpec(memory_space=pltpu.VMEM),
        out_shape=jax.ShapeDtypeStruct((_B // 128, 128), jnp.float32),
        scratch_shapes=[
            pltpu.VMEM((_K, _B), jnp.float32),
            pltpu.VMEM((_K, _B), jnp.float32),
            pltpu.SemaphoreType.DMA((2, _NCH)),
        ],
    )(gu.T, gi.T)
    return out.reshape(_B)


# speedup vs baseline: 6.4305x; 1.0214x over previous
import jax
import jax.numpy as jnp
from jax.experimental import pallas as pl
from jax.experimental.pallas import tpu as pltpu

_B, _K = 16384, 64
_NCH = 8
_CB = _B // _NCH


def _body(u_hbm, v_hbm, out_ref, u_v, v_v, sems):
    copies = []
    for c in range(_NCH):
        cu = pltpu.make_async_copy(
            u_hbm.at[:, pl.ds(c * _CB, _CB)],
            u_v.at[:, pl.ds(c * _CB, _CB)],
            sems.at[0, c],
        )
        cv = pltpu.make_async_copy(
            v_hbm.at[:, pl.ds(c * _CB, _CB)],
            v_v.at[:, pl.ds(c * _CB, _CB)],
            sems.at[1, c],
        )
        cu.start()
        cv.start()
        copies.append((cu, cv))
    for c in range(_NCH):
        cu, cv = copies[c]
        cu.wait()
        cv.wait()
        s = jnp.sum(
            u_v[:, pl.ds(c * _CB, _CB)] * v_v[:, pl.ds(c * _CB, _CB)], axis=0
        )
        out_ref[pl.ds(c * (_CB // 128), _CB // 128), :] = s.reshape(_CB // 128, 128)


def kernel(gu, gi):
    out = pl.pallas_call(
        _body,
        in_specs=[
            pl.BlockSpec(memory_space=pltpu.HBM),
            pl.BlockSpec(memory_space=pltpu.HBM),
        ],
        out_specs=pl.BlockSpec(memory_space=pltpu.VMEM),
        out_shape=jax.ShapeDtypeStruct((_B // 128, 128), jnp.float32),
        scratch_shapes=[
            pltpu.VMEM((_K, _B), jnp.float32),
            pltpu.VMEM((_K, _B), jnp.float32),
            pltpu.SemaphoreType.DMA((2, _NCH)),
        ],
    )(gu.T, gi.T)
    return out.reshape(_B)
